# baseline (speedup 1.0000x reference)
import functools

import jax
import jax.numpy as jnp
from jax.experimental import pallas as pl
from jax.experimental.pallas import tpu as pltpu

ROW_BLOCK = 256
SLAB = 768


def _fused_kernel(
    a_ref, o_ref, asc_ref, acc_ref, heavy_ref, *, n_row_blocks, k, recent, min_value
):
    s = pl.program_id(1)
    n = a_ref.shape[2]

    @pl.when(s < n_row_blocks)
    def _():
        a = a_ref[0]
        m = jnp.max(a, axis=1, keepdims=True)
        e = jnp.exp(a - m)
        ssum = jnp.sum(e, axis=1, keepdims=True)
        part = jnp.sum(e / ssum, axis=0, keepdims=True)

        @pl.when(s == 0)
        def _():
            acc_ref[...] = part

        @pl.when(s > 0)
        def _():
            acc_ref[...] = acc_ref[...] + part

        asc_ref[pl.ds(s * ROW_BLOCK, ROW_BLOCK), :] = a

    @pl.when(s == n_row_blocks)
    def _():
        vb = jax.lax.bitcast_convert_type(acc_ref[...], jnp.int32)

        def tsearch_body(b, t):
            cand = t | jax.lax.shift_left(jnp.int32(1), jnp.int32(30) - b)
            cnt = jnp.sum((vb >= cand).astype(jnp.int32))
            return jnp.where(cnt >= k, cand, t)

        t = jax.lax.fori_loop(0, 31, tsearch_body, jnp.int32(0))

        gt = vb > t
        eq = vb == t
        m_ties = k - jnp.sum(gt.astype(jnp.int32))
        idx = jax.lax.broadcasted_iota(jnp.int32, (1, n), 1)
        w = jnp.where(eq, idx, jnp.int32(2 * n))

        def isearch_body(b, x):
            cand = x & ~jax.lax.shift_left(jnp.int32(1), jnp.int32(11) - b)
            cnt = jnp.sum((w <= cand).astype(jnp.int32))
            return jnp.where(cnt >= m_ties, cand, x)

        x = jax.lax.fori_loop(0, 12, isearch_body, jnp.int32(4095))
        heavy_ref[...] = (gt | (eq & (idx <= x))).astype(jnp.int32)

        hv = heavy_ref[...] != 0
        minv = jnp.float32(min_value)

        def apply_body(r, _):
            r0 = r * ROW_BLOCK
            a = asc_ref[pl.ds(r0, ROW_BLOCK), :]
            o_ref[0, pl.ds(r0, ROW_BLOCK), :] = jnp.where(hv, a, minv)
            c0 = jnp.minimum(jnp.maximum(r0 - ROW_BLOCK, 0), n - SLAB)
            c0 = pl.multiple_of(c0, 128)
            i = (
                jax.lax.broadcasted_iota(jnp.int32, (ROW_BLOCK, SLAB), 0) + r0
            )
            j = (
                jax.lax.broadcasted_iota(jnp.int32, (ROW_BLOCK, SLAB), 1) + c0
            )
            band = (j <= i + recent) & (j >= i - recent)
            a_s = asc_ref[pl.ds(r0, ROW_BLOCK), pl.ds(c0, SLAB)]
            hv_s = heavy_ref[:, pl.ds(c0, SLAB)] != 0
            o_ref[0, pl.ds(r0, ROW_BLOCK), pl.ds(c0, SLAB)] = jnp.where(
                band | hv_s, a_s, minv
            )
            return 0

        jax.lax.fori_loop(0, n_row_blocks, apply_body, 0)


def kernel(attn_weights):
    bs, head, query_len, key_len = attn_weights.shape
    heavy_budget = min(int(0.1 * key_len), key_len)
    recent_budget = int(0.1 * key_len)
    min_value = float(jnp.finfo(attn_weights.dtype).min)

    a = attn_weights.reshape(bs * head, query_len, key_len)
    nh = bs * head
    n_row_blocks = query_len // ROW_BLOCK

    out = pl.pallas_call(
        functools.partial(
            _fused_kernel,
            n_row_blocks=n_row_blocks,
            k=heavy_budget,
            recent=recent_budget,
            min_value=min_value,
        ),
        grid=(nh, n_row_blocks + 1),
        in_specs=[
            pl.BlockSpec(
                (1, ROW_BLOCK, key_len),
                lambda h, s: (h, jnp.minimum(s, n_row_blocks - 1), 0),
            ),
        ],
        out_specs=pl.BlockSpec(
            (1, query_len, key_len), lambda h, s: (h, 0, 0)
        ),
        out_shape=jax.ShapeDtypeStruct((nh, query_len, key_len), jnp.float32),
        scratch_shapes=[
            pltpu.VMEM((query_len, key_len), jnp.float32),
            pltpu.VMEM((1, key_len), jnp.float32),
            pltpu.VMEM((1, key_len), jnp.int32),
        ],
    )(a)

    return out.reshape(bs, head, query_len, key_len)


# speedup vs baseline: 1.0093x; 1.0093x over previous
import functools

import jax
import jax.numpy as jnp
from jax.experimental import pallas as pl
from jax.experimental.pallas import tpu as pltpu

ROW_BLOCK = 256
SLAB = 768


def _fused_kernel(
    a_ref, o_ref, asc_ref, acc_ref, heavy_ref, *, n_row_blocks, k, recent, min_value
):
    s = pl.program_id(1)
    n = a_ref.shape[2]

    @pl.when(s < n_row_blocks)
    def _():
        a = a_ref[0]
        m = jnp.max(a, axis=1, keepdims=True)
        e = jnp.exp(a - m)
        ssum = jnp.sum(e, axis=1, keepdims=True)
        part = jnp.sum(e / ssum, axis=0, keepdims=True)

        @pl.when(s == 0)
        def _():
            acc_ref[...] = part

        @pl.when(s > 0)
        def _():
            acc_ref[...] = acc_ref[...] + part

        asc_ref[pl.ds(s * ROW_BLOCK, ROW_BLOCK), :] = a

    @pl.when(s == n_row_blocks)
    def _():
        vb = jax.lax.bitcast_convert_type(acc_ref[...], jnp.int32)

        def tsearch_body(b, t):
            cand = t | jax.lax.shift_left(jnp.int32(1), jnp.int32(30) - b)
            cnt = jnp.sum((vb >= cand).astype(jnp.int32))
            return jnp.where(cnt >= k, cand, t)

        t = jax.lax.fori_loop(0, 31, tsearch_body, jnp.int32(0))

        gt = vb > t
        eq = vb == t
        m_ties = k - jnp.sum(gt.astype(jnp.int32))
        idx = jax.lax.broadcasted_iota(jnp.int32, (1, n), 1)
        w = jnp.where(eq, idx, jnp.int32(2 * n))

        def isearch_body(b, x):
            cand = x & ~jax.lax.shift_left(jnp.int32(1), jnp.int32(11) - b)
            cnt = jnp.sum((w <= cand).astype(jnp.int32))
            return jnp.where(cnt >= m_ties, cand, x)

        x = jax.lax.fori_loop(0, 12, isearch_body, jnp.int32(4095))
        heavy_ref[...] = (gt | (eq & (idx <= x))).astype(jnp.int32)

        hv = heavy_ref[...] != 0
        minv = jnp.float32(min_value)

        def apply_body(r, _):
            r0 = r * ROW_BLOCK
            a = asc_ref[pl.ds(r0, ROW_BLOCK), :]
            o_ref[0, pl.ds(r0, ROW_BLOCK), :] = jnp.where(hv, a, minv)
            c0 = jnp.minimum(jnp.maximum(r0 - ROW_BLOCK, 0), n - SLAB)
            c0 = pl.multiple_of(c0, 128)
            i = (
                jax.lax.broadcasted_iota(jnp.int32, (ROW_BLOCK, SLAB), 0) + r0
            )
            j = (
                jax.lax.broadcasted_iota(jnp.int32, (ROW_BLOCK, SLAB), 1) + c0
            )
            band = (j <= i + recent) & (j >= i - recent)
            a_s = asc_ref[pl.ds(r0, ROW_BLOCK), pl.ds(c0, SLAB)]
            hv_s = heavy_ref[:, pl.ds(c0, SLAB)] != 0
            o_ref[0, pl.ds(r0, ROW_BLOCK), pl.ds(c0, SLAB)] = jnp.where(
                band | hv_s, a_s, minv
            )
            return 0

        jax.lax.fori_loop(0, n_row_blocks, apply_body, 0)


def kernel(attn_weights):
    bs, head, query_len, key_len = attn_weights.shape
    heavy_budget = min(int(0.1 * key_len), key_len)
    recent_budget = int(0.1 * key_len)
    min_value = float(jnp.finfo(attn_weights.dtype).min)

    a = attn_weights.reshape(bs * head, query_len, key_len)
    nh = bs * head
    n_row_blocks = query_len // ROW_BLOCK

    out = pl.pallas_call(
        functools.partial(
            _fused_kernel,
            n_row_blocks=n_row_blocks,
            k=heavy_budget,
            recent=recent_budget,
            min_value=min_value,
        ),
        grid=(nh, n_row_blocks + 1),
        in_specs=[
            pl.BlockSpec(
                (1, ROW_BLOCK, key_len),
                lambda h, s: (h, jnp.minimum(s, n_row_blocks - 1), 0),
            ),
        ],
        out_specs=pl.BlockSpec(
            (1, query_len, key_len), lambda h, s: (h, 0, 0)
        ),
        out_shape=jax.ShapeDtypeStruct((nh, query_len, key_len), jnp.float32),
        compiler_params=pltpu.CompilerParams(
            dimension_semantics=("parallel", "arbitrary")
        ),
        scratch_shapes=[
            pltpu.VMEM((query_len, key_len), jnp.float32),
            pltpu.VMEM((1, key_len), jnp.float32),
            pltpu.VMEM((1, key_len), jnp.int32),
        ],
    )(a)

    return out.reshape(bs, head, query_len, key_len)


# speedup vs baseline: 1.0255x; 1.0161x over previous
import functools

import jax
import jax.numpy as jnp
from jax.experimental import pallas as pl
from jax.experimental.pallas import tpu as pltpu

ROW_BLOCK = 256
SLAB = 768


def _fused_kernel(
    a_ref, o_ref, asc_ref, acc_ref, heavy_ref, *, n_row_blocks, k, recent, min_value
):
    s = pl.program_id(1)
    n = a_ref.shape[2]

    @pl.when(s < n_row_blocks)
    def _():
        a = a_ref[0]
        m = jnp.max(a, axis=1, keepdims=True)
        e = jnp.exp(a - m)
        ssum = jnp.sum(e, axis=1, keepdims=True)
        part = jnp.sum(e / ssum, axis=0, keepdims=True)

        @pl.when(s == 0)
        def _():
            acc_ref[...] = part

        @pl.when(s > 0)
        def _():
            acc_ref[...] = acc_ref[...] + part

        asc_ref[pl.ds(s * ROW_BLOCK, ROW_BLOCK), :] = a

    @pl.when(s == n_row_blocks)
    def _():
        vb = jax.lax.bitcast_convert_type(
            acc_ref[...].reshape(16, n // 16), jnp.int32
        )

        def tsearch_body(b, t):
            cand = t | jax.lax.shift_left(jnp.int32(1), jnp.int32(30) - b)
            cnt = jnp.sum((vb >= cand).astype(jnp.int32))
            return jnp.where(cnt >= k, cand, t)

        t = jax.lax.fori_loop(0, 31, tsearch_body, jnp.int32(0))

        gt = vb > t
        eq = vb == t
        m_ties = k - jnp.sum(gt.astype(jnp.int32))
        idx = (
            jax.lax.broadcasted_iota(jnp.int32, (16, n // 16), 0) * (n // 16)
            + jax.lax.broadcasted_iota(jnp.int32, (16, n // 16), 1)
        )
        w = jnp.where(eq, idx, jnp.int32(2 * n))

        def isearch_body(b, x):
            cand = x & ~jax.lax.shift_left(jnp.int32(1), jnp.int32(11) - b)
            cnt = jnp.sum((w <= cand).astype(jnp.int32))
            return jnp.where(cnt >= m_ties, cand, x)

        x = jax.lax.fori_loop(0, 12, isearch_body, jnp.int32(4095))
        heavy_ref[...] = (
            (gt | (eq & (idx <= x))).astype(jnp.int32).reshape(1, n)
        )

        hv = heavy_ref[...] != 0
        minv = jnp.float32(min_value)

        def apply_body(r, _):
            r0 = r * ROW_BLOCK
            a = asc_ref[pl.ds(r0, ROW_BLOCK), :]
            o_ref[0, pl.ds(r0, ROW_BLOCK), :] = jnp.where(hv, a, minv)
            c0 = jnp.minimum(jnp.maximum(r0 - ROW_BLOCK, 0), n - SLAB)
            c0 = pl.multiple_of(c0, 128)
            i = (
                jax.lax.broadcasted_iota(jnp.int32, (ROW_BLOCK, SLAB), 0) + r0
            )
            j = (
                jax.lax.broadcasted_iota(jnp.int32, (ROW_BLOCK, SLAB), 1) + c0
            )
            band = (j <= i + recent) & (j >= i - recent)
            a_s = asc_ref[pl.ds(r0, ROW_BLOCK), pl.ds(c0, SLAB)]
            hv_s = heavy_ref[:, pl.ds(c0, SLAB)] != 0
            o_ref[0, pl.ds(r0, ROW_BLOCK), pl.ds(c0, SLAB)] = jnp.where(
                band | hv_s, a_s, minv
            )
            return 0

        jax.lax.fori_loop(0, n_row_blocks, apply_body, 0)


def kernel(attn_weights):
    bs, head, query_len, key_len = attn_weights.shape
    heavy_budget = min(int(0.1 * key_len), key_len)
    recent_budget = int(0.1 * key_len)
    min_value = float(jnp.finfo(attn_weights.dtype).min)

    a = attn_weights.reshape(bs * head, query_len, key_len)
    nh = bs * head
    n_row_blocks = query_len // ROW_BLOCK

    out = pl.pallas_call(
        functools.partial(
            _fused_kernel,
            n_row_blocks=n_row_blocks,
            k=heavy_budget,
            recent=recent_budget,
            min_value=min_value,
        ),
        grid=(nh, n_row_blocks + 1),
        in_specs=[
            pl.BlockSpec(
                (1, ROW_BLOCK, key_len),
                lambda h, s: (h, jnp.minimum(s, n_row_blocks - 1), 0),
            ),
        ],
        out_specs=pl.BlockSpec(
            (1, query_len, key_len), lambda h, s: (h, 0, 0)
        ),
        out_shape=jax.ShapeDtypeStruct((nh, query_len, key_len), jnp.float32),
        scratch_shapes=[
            pltpu.VMEM((query_len, key_len), jnp.float32),
            pltpu.VMEM((1, key_len), jnp.float32),
            pltpu.VMEM((1, key_len), jnp.int32),
        ],
    )(a)

    return out.reshape(bs, head, query_len, key_len)


# speedup vs baseline: 1.0269x; 1.0014x over previous
import functools

import jax
import jax.numpy as jnp
from jax.experimental import pallas as pl
from jax.experimental.pallas import tpu as pltpu

ROW_BLOCK = 256
SLAB = 768


def _fused_kernel(
    a_ref, o_ref, asc_ref, acc_ref, heavy_ref, *, n_row_blocks, k, recent, min_value
):
    s = pl.program_id(1)
    n = a_ref.shape[2]

    @pl.when(s < n_row_blocks)
    def _():
        a = a_ref[0]
        m = jnp.max(a, axis=1, keepdims=True)
        e = jnp.exp(a - m)
        ssum = jnp.sum(e, axis=1, keepdims=True)
        part = jnp.sum(e / ssum, axis=0, keepdims=True)

        @pl.when(s == 0)
        def _():
            acc_ref[...] = part

        @pl.when(s > 0)
        def _():
            acc_ref[...] = acc_ref[...] + part

        asc_ref[pl.ds(s * ROW_BLOCK, ROW_BLOCK), :] = a

    @pl.when(s == n_row_blocks)
    def _():
        vb = jax.lax.bitcast_convert_type(
            acc_ref[...].reshape(16, n // 16), jnp.int32
        )

        def tsearch_body(b, t):
            cand = t | jax.lax.shift_left(jnp.int32(1), jnp.int32(30) - b)
            cnt = jnp.sum(
                (vb >= cand).astype(jnp.int32), axis=(0, 1), keepdims=True
            )
            return jnp.where(cnt >= k, cand, t)

        t = jax.lax.fori_loop(0, 31, tsearch_body, jnp.zeros((1, 1), jnp.int32))

        gt = vb > t
        eq = vb == t
        m_ties = k - jnp.sum(gt.astype(jnp.int32), axis=(0, 1), keepdims=True)
        idx = (
            jax.lax.broadcasted_iota(jnp.int32, (16, n // 16), 0) * (n // 16)
            + jax.lax.broadcasted_iota(jnp.int32, (16, n // 16), 1)
        )
        w = jnp.where(eq, idx, jnp.int32(2 * n))

        def isearch_body(b, x):
            cand = x & ~jax.lax.shift_left(jnp.int32(1), jnp.int32(11) - b)
            cnt = jnp.sum(
                (w <= cand).astype(jnp.int32), axis=(0, 1), keepdims=True
            )
            return jnp.where(cnt >= m_ties, cand, x)

        x = jax.lax.fori_loop(
            0, 12, isearch_body, jnp.full((1, 1), 4095, jnp.int32)
        )
        heavy_ref[...] = (
            (gt | (eq & (idx <= x))).astype(jnp.int32).reshape(1, n)
        )

        hv = heavy_ref[...] != 0
        minv = jnp.float32(min_value)

        def apply_body(r, _):
            r0 = r * ROW_BLOCK
            a = asc_ref[pl.ds(r0, ROW_BLOCK), :]
            o_ref[0, pl.ds(r0, ROW_BLOCK), :] = jnp.where(hv, a, minv)
            c0 = jnp.minimum(jnp.maximum(r0 - ROW_BLOCK, 0), n - SLAB)
            c0 = pl.multiple_of(c0, 128)
            i = (
                jax.lax.broadcasted_iota(jnp.int32, (ROW_BLOCK, SLAB), 0) + r0
            )
            j = (
                jax.lax.broadcasted_iota(jnp.int32, (ROW_BLOCK, SLAB), 1) + c0
            )
            band = (j <= i + recent) & (j >= i - recent)
            a_s = asc_ref[pl.ds(r0, ROW_BLOCK), pl.ds(c0, SLAB)]
            hv_s = heavy_ref[:, pl.ds(c0, SLAB)] != 0
            o_ref[0, pl.ds(r0, ROW_BLOCK), pl.ds(c0, SLAB)] = jnp.where(
                band | hv_s, a_s, minv
            )
            return 0

        jax.lax.fori_loop(0, n_row_blocks, apply_body, 0)


def kernel(attn_weights):
    bs, head, query_len, key_len = attn_weights.shape
    heavy_budget = min(int(0.1 * key_len), key_len)
    recent_budget = int(0.1 * key_len)
    min_value = float(jnp.finfo(attn_weights.dtype).min)

    a = attn_weights.reshape(bs * head, query_len, key_len)
    nh = bs * head
    n_row_blocks = query_len // ROW_BLOCK

    out = pl.pallas_call(
        functools.partial(
            _fused_kernel,
            n_row_blocks=n_row_blocks,
            k=heavy_budget,
            recent=recent_budget,
            min_value=min_value,
        ),
        grid=(nh, n_row_blocks + 1),
        in_specs=[
            pl.BlockSpec(
                (1, ROW_BLOCK, key_len),
                lambda h, s: (h, jnp.minimum(s, n_row_blocks - 1), 0),
            ),
        ],
        out_specs=pl.BlockSpec(
            (1, query_len, key_len), lambda h, s: (h, 0, 0)
        ),
        out_shape=jax.ShapeDtypeStruct((nh, query_len, key_len), jnp.float32),
        scratch_shapes=[
            pltpu.VMEM((query_len, key_len), jnp.float32),
            pltpu.VMEM((1, key_len), jnp.float32),
            pltpu.VMEM((1, key_len), jnp.int32),
        ],
    )(a)

    return out.reshape(bs, head, query_len, key_len)


# speedup vs baseline: 1.2476x; 1.2150x over previous
import functools

import jax
import jax.numpy as jnp
from jax.experimental import pallas as pl
from jax.experimental.pallas import tpu as pltpu

ROW_BLOCK = 256
SLAB = 768


def _fused_kernel(
    a_ref, o_ref, asc_ref, acc_ref, heavy_ref, *, n_row_blocks, k, recent, min_value
):
    s = pl.program_id(1)
    n = a_ref.shape[2]

    @pl.when(s < n_row_blocks)
    def _():
        a = a_ref[0]
        m = jnp.max(a, axis=1, keepdims=True)
        e = jnp.exp(a - m)
        ssum = jnp.sum(e, axis=1, keepdims=True)
        part = jnp.sum(e / ssum, axis=0, keepdims=True)

        @pl.when(s == 0)
        def _():
            acc_ref[...] = part

        @pl.when(s > 0)
        def _():
            acc_ref[...] = acc_ref[...] + part

        asc_ref[pl.ds(s * ROW_BLOCK, ROW_BLOCK), :] = a

    @pl.when(s == n_row_blocks)
    def _():
        vb = jax.lax.bitcast_convert_type(acc_ref[...], jnp.int32)
        mvec = jax.lax.broadcasted_iota(jnp.int32, (8, 1), 0)

        def radix_max_search(arr, need, shifts_bits, width):
            t = jnp.zeros((1, 1), jnp.int32)
            for shift, nbits in shifts_bits:
                mv = mvec & ((1 << nbits) - 1)
                cand = t | jax.lax.shift_left(mv, shift)
                cnt = jnp.sum(
                    (arr >= cand).astype(jnp.int32), axis=1, keepdims=True
                )
                ok = cnt >= need
                mstar = jnp.max(
                    jnp.where(ok, mv, 0), axis=0, keepdims=True
                )
                t = t | jax.lax.shift_left(mstar, shift)
            return t

        t_rounds = [(sh, 3) for sh in range(28, 0, -3)] + [(0, 1)]
        t = radix_max_search(vb, jnp.full((1, 1), k, jnp.int32), t_rounds, 31)

        gt = vb > t
        eq = vb == t
        m_ties = k - jnp.sum(gt.astype(jnp.int32), axis=1, keepdims=True)
        idx = jax.lax.broadcasted_iota(jnp.int32, (1, n), 1)
        u = jnp.where(eq, jnp.int32(4095) - idx, jnp.int32(-1))
        u_rounds = [(9, 3), (6, 3), (3, 3), (0, 3)]
        ucut = radix_max_search(u, m_ties, u_rounds, 12)
        heavy_ref[...] = (gt | (eq & (u >= ucut))).astype(jnp.int32)

        hv = heavy_ref[...] != 0
        minv = jnp.float32(min_value)

        def apply_body(r, _):
            r0 = r * ROW_BLOCK
            a = asc_ref[pl.ds(r0, ROW_BLOCK), :]
            o_ref[0, pl.ds(r0, ROW_BLOCK), :] = jnp.where(hv, a, minv)
            c0 = jnp.minimum(jnp.maximum(r0 - ROW_BLOCK, 0), n - SLAB)
            c0 = pl.multiple_of(c0, 128)
            i = (
                jax.lax.broadcasted_iota(jnp.int32, (ROW_BLOCK, SLAB), 0) + r0
            )
            j = (
                jax.lax.broadcasted_iota(jnp.int32, (ROW_BLOCK, SLAB), 1) + c0
            )
            band = (j <= i + recent) & (j >= i - recent)
            a_s = asc_ref[pl.ds(r0, ROW_BLOCK), pl.ds(c0, SLAB)]
            hv_s = heavy_ref[:, pl.ds(c0, SLAB)] != 0
            o_ref[0, pl.ds(r0, ROW_BLOCK), pl.ds(c0, SLAB)] = jnp.where(
                band | hv_s, a_s, minv
            )
            return 0

        jax.lax.fori_loop(0, n_row_blocks, apply_body, 0)


def kernel(attn_weights):
    bs, head, query_len, key_len = attn_weights.shape
    heavy_budget = min(int(0.1 * key_len), key_len)
    recent_budget = int(0.1 * key_len)
    min_value = float(jnp.finfo(attn_weights.dtype).min)

    a = attn_weights.reshape(bs * head, query_len, key_len)
    nh = bs * head
    n_row_blocks = query_len // ROW_BLOCK

    out = pl.pallas_call(
        functools.partial(
            _fused_kernel,
            n_row_blocks=n_row_blocks,
            k=heavy_budget,
            recent=recent_budget,
            min_value=min_value,
        ),
        grid=(nh, n_row_blocks + 1),
        in_specs=[
            pl.BlockSpec(
                (1, ROW_BLOCK, key_len),
                lambda h, s: (h, jnp.minimum(s, n_row_blocks - 1), 0),
            ),
        ],
        out_specs=pl.BlockSpec(
            (1, query_len, key_len), lambda h, s: (h, 0, 0)
        ),
        out_shape=jax.ShapeDtypeStruct((nh, query_len, key_len), jnp.float32),
        scratch_shapes=[
            pltpu.VMEM((query_len, key_len), jnp.float32),
            pltpu.VMEM((1, key_len), jnp.float32),
            pltpu.VMEM((1, key_len), jnp.int32),
        ],
    )(a)

    return out.reshape(bs, head, query_len, key_len)


# speedup vs baseline: 1.2651x; 1.0140x over previous
import functools

import jax
import jax.numpy as jnp
from jax.experimental import pallas as pl
from jax.experimental.pallas import tpu as pltpu

ROW_BLOCK = 256
SLAB = 768


def _fused_kernel(
    a_ref, o_ref, asc_ref, acc_ref, heavy_ref, *, n_row_blocks, k, recent, min_value
):
    s = pl.program_id(1)
    n = a_ref.shape[2]

    @pl.when(s < n_row_blocks)
    def _():
        a = a_ref[0]
        m = jnp.max(a, axis=1, keepdims=True)
        e = jnp.exp(a - m)
        ssum = jnp.sum(e, axis=1, keepdims=True)
        part = jnp.sum(e / ssum, axis=0, keepdims=True)

        @pl.when(s == 0)
        def _():
            acc_ref[...] = part

        @pl.when(s > 0)
        def _():
            acc_ref[...] = acc_ref[...] + part

        asc_ref[pl.ds(s * ROW_BLOCK, ROW_BLOCK), :] = a

    @pl.when(s == n_row_blocks)
    def _():
        vb = jax.lax.bitcast_convert_type(acc_ref[...], jnp.int32)
        mvec = jax.lax.broadcasted_iota(jnp.int32, (16, 1), 0)

        def radix_max_search(arr, need, shifts_bits, width):
            t = jnp.zeros((1, 1), jnp.int32)
            for shift, nbits in shifts_bits:
                mv = mvec & ((1 << nbits) - 1)
                cand = t | jax.lax.shift_left(mv, shift)
                cnt = jnp.sum(
                    (arr >= cand).astype(jnp.int32), axis=1, keepdims=True
                )
                ok = cnt >= need
                mstar = jnp.max(
                    jnp.where(ok, mv, 0), axis=0, keepdims=True
                )
                t = t | jax.lax.shift_left(mstar, shift)
            return t

        t_rounds = [(sh, 4) for sh in range(27, 0, -4)] + [(0, 3)]
        t = radix_max_search(vb, jnp.full((1, 1), k, jnp.int32), t_rounds, 31)

        gt = vb > t
        eq = vb == t
        m_ties = k - jnp.sum(gt.astype(jnp.int32), axis=1, keepdims=True)
        idx = jax.lax.broadcasted_iota(jnp.int32, (1, n), 1)
        u = jnp.where(eq, jnp.int32(4095) - idx, jnp.int32(-1))
        u_rounds = [(8, 4), (4, 4), (0, 4)]
        ucut = radix_max_search(u, m_ties, u_rounds, 12)
        heavy_ref[...] = (gt | (eq & (u >= ucut))).astype(jnp.int32)

        hv = heavy_ref[...] != 0
        minv = jnp.float32(min_value)

        def apply_body(r, _):
            r0 = r * ROW_BLOCK
            a = asc_ref[pl.ds(r0, ROW_BLOCK), :]
            o_ref[0, pl.ds(r0, ROW_BLOCK), :] = jnp.where(hv, a, minv)
            c0 = jnp.minimum(jnp.maximum(r0 - ROW_BLOCK, 0), n - SLAB)
            c0 = pl.multiple_of(c0, 128)
            i = (
                jax.lax.broadcasted_iota(jnp.int32, (ROW_BLOCK, SLAB), 0) + r0
            )
            j = (
                jax.lax.broadcasted_iota(jnp.int32, (ROW_BLOCK, SLAB), 1) + c0
            )
            band = (j <= i + recent) & (j >= i - recent)
            a_s = asc_ref[pl.ds(r0, ROW_BLOCK), pl.ds(c0, SLAB)]
            hv_s = heavy_ref[:, pl.ds(c0, SLAB)] != 0
            o_ref[0, pl.ds(r0, ROW_BLOCK), pl.ds(c0, SLAB)] = jnp.where(
                band | hv_s, a_s, minv
            )
            return 0

        jax.lax.fori_loop(0, n_row_blocks, apply_body, 0)


def kernel(attn_weights):
    bs, head, query_len, key_len = attn_weights.shape
    heavy_budget = min(int(0.1 * key_len), key_len)
    recent_budget = int(0.1 * key_len)
    min_value = float(jnp.finfo(attn_weights.dtype).min)

    a = attn_weights.reshape(bs * head, query_len, key_len)
    nh = bs * head
    n_row_blocks = query_len // ROW_BLOCK

    out = pl.pallas_call(
        functools.partial(
            _fused_kernel,
            n_row_blocks=n_row_blocks,
            k=heavy_budget,
            recent=recent_budget,
            min_value=min_value,
        ),
        grid=(nh, n_row_blocks + 1),
        in_specs=[
            pl.BlockSpec(
                (1, ROW_BLOCK, key_len),
                lambda h, s: (h, jnp.minimum(s, n_row_blocks - 1), 0),
            ),
        ],
        out_specs=pl.BlockSpec(
            (1, query_len, key_len), lambda h, s: (h, 0, 0)
        ),
        out_shape=jax.ShapeDtypeStruct((nh, query_len, key_len), jnp.float32),
        scratch_shapes=[
            pltpu.VMEM((query_len, key_len), jnp.float32),
            pltpu.VMEM((1, key_len), jnp.float32),
            pltpu.VMEM((1, key_len), jnp.int32),
        ],
    )(a)

    return out.reshape(bs, head, query_len, key_len)
